# Initial kernel scaffold; baseline (speedup 1.0000x reference)
import jax
import jax.numpy as jnp
from jax.experimental import pallas as pl


def kernel(logits, T, penalties, Qhat):
    raise NotImplementedError("write your pallas kernel here")


# speedup vs baseline: 4.5038x; 4.5038x over previous
import jax
import jax.numpy as jnp
from jax import lax
from jax.experimental import pallas as pl

K = 128
ROWS_PER_BLK = 8


def _stats_body(x_ref, t_ref, m_ref, z_ref):
    x = x_ref[...]
    y = x / t_ref[...]
    m = jnp.max(y, axis=1, keepdims=True)
    e = jnp.exp(y - m)
    z = jnp.sum(e, axis=1, keepdims=True)
    m_ref[...] = m
    z_ref[...] = z


def _thresh_body(v_ref, i_ref, m_ref, z_ref, pc_ref, t_ref, q_ref,
                 tout_ref, bout_ref):
    v = v_ref[...]
    y = v / t_ref[...]
    s = jnp.exp(y - m_ref[...]) / z_ref[...]
    c = s
    d = 1
    while d < K:
        c = c + jnp.concatenate(
            [jnp.zeros((d, c.shape[1]), c.dtype), c[:K - d, :]], axis=0)
        d *= 2
    f = c + pc_ref[...]
    cnt = jnp.sum((f <= q_ref[...]).astype(jnp.int32), axis=0, keepdims=True)
    sizes = jnp.minimum(cnt + 1, K)
    row = lax.broadcasted_iota(jnp.int32, v.shape, 0)
    oh = row == (sizes - 1)
    tout_ref[...] = jnp.sum(jnp.where(oh, v, 0.0), axis=0, keepdims=True)
    bout_ref[...] = jnp.sum(jnp.where(oh, i_ref[...], 0), axis=0, keepdims=True)


def _mask_body(x_ref, t_ref, b_ref, mask_ref, copy_ref):
    x = x_ref[...]
    t = t_ref[...]
    b = b_ref[...]
    col = lax.broadcasted_iota(jnp.int32, x.shape, 1)
    keep = (x > t) | ((x == t) & (col <= b))
    mask_ref[...] = keep.astype(jnp.float32)
    copy_ref[...] = x


def _topk(logits):
    return jax.lax.top_k(logits, K)


def kernel(logits, T, penalties, Qhat):
    B, C = logits.shape
    f32 = jnp.float32
    T2 = jnp.asarray(T, f32).reshape(1, 1)
    Q2 = jnp.asarray(Qhat, f32).reshape(1, 1)
    nblk = B // ROWS_PER_BLK

    m, Z = pl.pallas_call(
        _stats_body,
        grid=(nblk,),
        in_specs=[pl.BlockSpec((ROWS_PER_BLK, C), lambda i: (i, 0)),
                  pl.BlockSpec((1, 1), lambda i: (0, 0))],
        out_specs=[pl.BlockSpec((ROWS_PER_BLK, 1), lambda i: (i, 0)),
                   pl.BlockSpec((ROWS_PER_BLK, 1), lambda i: (i, 0))],
        out_shape=[jax.ShapeDtypeStruct((B, 1), f32),
                   jax.ShapeDtypeStruct((B, 1), f32)],
    )(logits, T2)

    topv, topi = _topk(logits)

    pcK = jnp.cumsum(penalties, axis=1)[:, :K].T

    tT, bT = pl.pallas_call(
        _thresh_body,
        out_shape=[jax.ShapeDtypeStruct((1, B), f32),
                   jax.ShapeDtypeStruct((1, B), jnp.int32)],
    )(topv.T, topi.T, m.T, Z.T, pcK, T2, Q2)

    mask, xcopy = pl.pallas_call(
        _mask_body,
        grid=(nblk,),
        in_specs=[pl.BlockSpec((ROWS_PER_BLK, C), lambda i: (i, 0)),
                  pl.BlockSpec((ROWS_PER_BLK, 1), lambda i: (i, 0)),
                  pl.BlockSpec((ROWS_PER_BLK, 1), lambda i: (i, 0))],
        out_specs=[pl.BlockSpec((ROWS_PER_BLK, C), lambda i: (i, 0)),
                   pl.BlockSpec((ROWS_PER_BLK, C), lambda i: (i, 0))],
        out_shape=[jax.ShapeDtypeStruct((B, C), f32),
                   jax.ShapeDtypeStruct((B, C), f32)],
    )(logits, tT.T, bT.T)

    return (xcopy, mask)


# speedup vs baseline: 17.3043x; 3.8421x over previous
import functools

import jax
import jax.numpy as jnp
from jax import lax
from jax.experimental import pallas as pl
from jax.experimental.pallas import tpu as pltpu
from jax.experimental.pallas import tpu_sc as plsc

K = 128
ROWS_PER_BLK = 8

SC_CORES = 2
SC_SUBCORES = 16
NW = SC_CORES * SC_SUBCORES
CH = 10000
CAP = 26624
OUTCAP = 256
RESEL_TRIG = 4096
BISECT_ITERS = 26


def _stats_body(x_ref, t_ref, m_ref, z_ref):
    x = x_ref[...]
    y = x / t_ref[...]
    m = jnp.max(y, axis=1, keepdims=True)
    e = jnp.exp(y - m)
    z = jnp.sum(e, axis=1, keepdims=True)
    m_ref[...] = m
    z_ref[...] = z


def _sc_select(logits):
    B, C = logits.shape
    rows_per_w = B // NW
    npairs = C // (2 * CH)
    mesh = plsc.VectorSubcoreMesh(
        core_axis_name="c", subcore_axis_name="s",
        num_cores=SC_CORES, num_subcores=SC_SUBCORES)
    neg_inf = jnp.float32(-jnp.inf)

    @functools.partial(
        pl.kernel,
        out_type=[jax.ShapeDtypeStruct((B * OUTCAP,), jnp.float32),
                  jax.ShapeDtypeStruct((B * OUTCAP,), jnp.int32)],
        mesh=mesh,
        compiler_params=pltpu.CompilerParams(needs_layout_passes=False),
        scratch_types=[
            pltpu.VMEM((2 * CH,), jnp.float32),
            pltpu.VMEM((CAP,), jnp.float32),
            pltpu.VMEM((CAP,), jnp.int32),
            pltpu.VMEM((OUTCAP,), jnp.float32),
            pltpu.VMEM((OUTCAP,), jnp.int32),
            pltpu.SemaphoreType.DMA,
            pltpu.SemaphoreType.DMA,
        ],
    )
    def sel(x_hbm, outv_hbm, outi_hbm, buf, cv, ci, sv, si, sem0, sem1):
        wid = lax.axis_index("s") * SC_CORES + lax.axis_index("c")
        lane = lax.iota(jnp.int32, 16)

        def count_gt(u, nc):
            nv = (nc + 15) // 16
            def cb(j, acc):
                v = cv[pl.ds(j * 16, 16)]
                valid = (j * 16 + lane) < nc
                return acc + jnp.where((v > u) & valid, 1, 0)
            acc = lax.fori_loop(0, nv, cb, jnp.zeros((16,), jnp.int32))
            return jnp.sum(acc)

        def bisect(nc):
            def bb(_, lohi):
                lo, hi = lohi
                mid = jnp.float32(0.5) * (lo + hi)
                good = count_gt(mid, nc) >= K
                return (jnp.where(good, mid, lo), jnp.where(good, hi, mid))
            lo, _ = lax.fori_loop(0, BISECT_ITERS, bb,
                                  (jnp.float32(-64.0), jnp.float32(64.0)))
            return lo

        def reselect(thr, nc):
            nv = (nc + 15) // 16
            def rb(j, run):
                v = cv[pl.ds(j * 16, 16)]
                ii = ci[pl.ds(j * 16, 16)]
                valid = (j * 16 + lane) < nc
                msk = (v > thr) & valid
                mi = msk.astype(jnp.int32)
                pc = plsc.cumsum(mi)
                pos = run + (pc - mi)
                plsc.store_scatter(cv, [pos], v, mask=msk)
                plsc.store_scatter(ci, [pos], ii, mask=msk)
                return run + jnp.max(pc)
            run = lax.fori_loop(0, nv, rb, jnp.zeros((16,), jnp.int32))
            return jnp.max(run)

        def maybe_retighten(thr, nc, trig):
            def yes(args):
                thr0, nc0 = args
                lo = jnp.maximum(thr0, bisect(nc0))
                return lo, reselect(lo, nc0)
            return lax.cond(nc > trig, yes, lambda a: a, (thr, nc))

        def process_chunk(base_off, col0, thr, nc):
            def fb(i, run):
                v = buf[pl.ds(base_off + i * 16, 16)]
                msk = v > thr
                mi = msk.astype(jnp.int32)
                pc = plsc.cumsum(mi)
                pos = jnp.minimum(run + (pc - mi), CAP - 1)
                plsc.store_scatter(cv, [pos], v, mask=msk)
                plsc.store_scatter(ci, [pos], col0 + i * 16 + lane, mask=msk)
                return run + jnp.max(pc)
            run0 = jnp.zeros((16,), jnp.int32) + nc
            run = lax.fori_loop(0, CH // 16, fb, run0)
            return jnp.max(run)

        def row_fn(q, _):
            row = wid * rows_per_w + q
            rbase = row * C
            pltpu.make_async_copy(
                x_hbm.at[pl.ds(rbase, CH)], buf.at[pl.ds(0, CH)], sem0).start()
            thr = neg_inf
            nc = jnp.int32(0)

            def pair_body(p, carry):
                thr, nc = carry
                k0 = 2 * p
                pltpu.make_async_copy(
                    x_hbm.at[pl.ds(rbase + (k0 + 1) * CH, CH)],
                    buf.at[pl.ds(CH, CH)], sem1).start()
                pltpu.make_async_copy(
                    x_hbm.at[pl.ds(rbase + k0 * CH, CH)],
                    buf.at[pl.ds(0, CH)], sem0).wait()
                nc = process_chunk(0, k0 * CH, thr, nc)
                thr, nc = maybe_retighten(thr, nc, RESEL_TRIG)

                @pl.when(p < npairs - 1)
                def _():
                    pltpu.make_async_copy(
                        x_hbm.at[pl.ds(rbase + (k0 + 2) * CH, CH)],
                        buf.at[pl.ds(0, CH)], sem0).start()
                pltpu.make_async_copy(
                    x_hbm.at[pl.ds(rbase + (k0 + 1) * CH, CH)],
                    buf.at[pl.ds(CH, CH)], sem1).wait()
                nc = process_chunk(CH, (k0 + 1) * CH, thr, nc)
                thr, nc = maybe_retighten(thr, nc, RESEL_TRIG)
                return thr, nc

            thr, nc = lax.fori_loop(0, npairs, pair_body, (thr, nc))
            thr, nc = maybe_retighten(thr, nc, OUTCAP)
            ncf = jnp.minimum(nc, OUTCAP)
            for t in range(OUTCAP // 16):
                g = (t * 16 + lane) < ncf
                sv[pl.ds(t * 16, 16)] = jnp.where(g, cv[pl.ds(t * 16, 16)],
                                                  neg_inf)
                si[pl.ds(t * 16, 16)] = jnp.where(g, ci[pl.ds(t * 16, 16)], 0)
            pltpu.sync_copy(sv, outv_hbm.at[pl.ds(row * OUTCAP, OUTCAP)])
            pltpu.sync_copy(si, outi_hbm.at[pl.ds(row * OUTCAP, OUTCAP)])
            return 0

        lax.fori_loop(0, rows_per_w, row_fn, 0)

    fv, fi = sel(logits.reshape(-1))
    return fv.reshape(B, OUTCAP), fi.reshape(B, OUTCAP)


def _thresh_body(v_ref, i_ref, m_ref, z_ref, pc_ref, t_ref, q_ref,
                 tout_ref, bout_ref):
    v = v_ref[...]
    ii = i_ref[...]
    gt = (v[:, None, :] > v[:, :, None]) | (
        (v[:, None, :] == v[:, :, None]) & (ii[:, None, :] < ii[:, :, None]))
    rank = jnp.sum(gt.astype(jnp.int32), axis=2)
    oh = rank[:, :, None] == lax.broadcasted_iota(
        jnp.int32, (v.shape[0], OUTCAP, K), 2)
    sv = jnp.sum(jnp.where(oh, v[:, :, None], 0.0), axis=1)
    si = jnp.sum(jnp.where(oh, ii[:, :, None], 0), axis=1)

    y = sv / t_ref[...]
    s = jnp.exp(y - m_ref[...]) / z_ref[...]
    c = s
    d = 1
    while d < K:
        c = c + jnp.concatenate(
            [jnp.zeros((c.shape[0], d), c.dtype), c[:, :K - d]], axis=1)
        d *= 2
    f = c + pc_ref[...]
    cnt = jnp.sum((f <= q_ref[...]).astype(jnp.int32), axis=1, keepdims=True)
    sizes = jnp.minimum(cnt + 1, K)
    col = lax.broadcasted_iota(jnp.int32, sv.shape, 1)
    oh2 = col == (sizes - 1)
    tout_ref[...] = jnp.sum(jnp.where(oh2, sv, 0.0), axis=1, keepdims=True)
    bout_ref[...] = jnp.sum(jnp.where(oh2, si, 0), axis=1, keepdims=True)


def _mask_body(x_ref, t_ref, b_ref, mask_ref, copy_ref):
    x = x_ref[...]
    t = t_ref[...]
    b = b_ref[...]
    col = lax.broadcasted_iota(jnp.int32, x.shape, 1)
    keep = (x > t) | ((x == t) & (col <= b))
    mask_ref[...] = keep.astype(jnp.float32)
    copy_ref[...] = x


def kernel(logits, T, penalties, Qhat):
    B, C = logits.shape
    f32 = jnp.float32
    T2 = jnp.asarray(T, f32).reshape(1, 1)
    Q2 = jnp.asarray(Qhat, f32).reshape(1, 1)
    nblk = B // ROWS_PER_BLK

    m, Z = pl.pallas_call(
        _stats_body,
        grid=(nblk,),
        in_specs=[pl.BlockSpec((ROWS_PER_BLK, C), lambda i: (i, 0)),
                  pl.BlockSpec((1, 1), lambda i: (0, 0))],
        out_specs=[pl.BlockSpec((ROWS_PER_BLK, 1), lambda i: (i, 0)),
                   pl.BlockSpec((ROWS_PER_BLK, 1), lambda i: (i, 0))],
        out_shape=[jax.ShapeDtypeStruct((B, 1), f32),
                   jax.ShapeDtypeStruct((B, 1), f32)],
    )(logits, T2)

    candv, candi = _sc_select(logits)

    pcK = jnp.cumsum(penalties, axis=1)[:, :K]

    tcol, bcol = pl.pallas_call(
        _thresh_body,
        grid=(nblk,),
        in_specs=[pl.BlockSpec((ROWS_PER_BLK, OUTCAP), lambda i: (i, 0)),
                  pl.BlockSpec((ROWS_PER_BLK, OUTCAP), lambda i: (i, 0)),
                  pl.BlockSpec((ROWS_PER_BLK, 1), lambda i: (i, 0)),
                  pl.BlockSpec((ROWS_PER_BLK, 1), lambda i: (i, 0)),
                  pl.BlockSpec((1, K), lambda i: (0, 0)),
                  pl.BlockSpec((1, 1), lambda i: (0, 0)),
                  pl.BlockSpec((1, 1), lambda i: (0, 0))],
        out_specs=[pl.BlockSpec((ROWS_PER_BLK, 1), lambda i: (i, 0)),
                   pl.BlockSpec((ROWS_PER_BLK, 1), lambda i: (i, 0))],
        out_shape=[jax.ShapeDtypeStruct((B, 1), f32),
                   jax.ShapeDtypeStruct((B, 1), jnp.int32)],
    )(candv, candi, m, Z, pcK, T2, Q2)

    mask, xcopy = pl.pallas_call(
        _mask_body,
        grid=(nblk,),
        in_specs=[pl.BlockSpec((ROWS_PER_BLK, C), lambda i: (i, 0)),
                  pl.BlockSpec((ROWS_PER_BLK, 1), lambda i: (i, 0)),
                  pl.BlockSpec((ROWS_PER_BLK, 1), lambda i: (i, 0))],
        out_specs=[pl.BlockSpec((ROWS_PER_BLK, C), lambda i: (i, 0)),
                   pl.BlockSpec((ROWS_PER_BLK, C), lambda i: (i, 0))],
        out_shape=[jax.ShapeDtypeStruct((B, C), f32),
                   jax.ShapeDtypeStruct((B, C), f32)],
    )(logits, tcol, bcol)

    return (xcopy, mask)


# speedup vs baseline: 21.9904x; 1.2708x over previous
import functools

import jax
import jax.numpy as jnp
from jax import lax
from jax.experimental import pallas as pl
from jax.experimental.pallas import tpu as pltpu
from jax.experimental.pallas import tpu_sc as plsc

K = 128
ROWS_PER_BLK = 8

SC_CORES = 2
SC_SUBCORES = 16
NW = SC_CORES * SC_SUBCORES
CH = 10000
CAP = 26624
OUTCAP = 256
RESEL_TRIG = 4096
BISECT_ITERS = 26
WIN = 2048
UNROLL = 5


def _stats_body(x_ref, t_ref, m_ref, z_ref):
    x = x_ref[...]
    y = x / t_ref[...]
    m = jnp.max(y, axis=1, keepdims=True)
    e = jnp.exp(y - m)
    z = jnp.sum(e, axis=1, keepdims=True)
    m_ref[...] = m
    z_ref[...] = z


def _sc_select(logits):
    B, C = logits.shape
    rows_per_w = B // NW
    npairs = C // (2 * CH)
    mesh = plsc.VectorSubcoreMesh(
        core_axis_name="c", subcore_axis_name="s",
        num_cores=SC_CORES, num_subcores=SC_SUBCORES)
    neg_inf = jnp.float32(-jnp.inf)

    @functools.partial(
        pl.kernel,
        out_type=[jax.ShapeDtypeStruct((B * OUTCAP,), jnp.float32),
                  jax.ShapeDtypeStruct((B * OUTCAP,), jnp.int32)],
        mesh=mesh,
        compiler_params=pltpu.CompilerParams(needs_layout_passes=False),
        scratch_types=[
            pltpu.VMEM((2 * CH,), jnp.float32),
            pltpu.VMEM((CAP,), jnp.float32),
            pltpu.VMEM((CAP,), jnp.int32),
            pltpu.VMEM((OUTCAP,), jnp.float32),
            pltpu.VMEM((OUTCAP,), jnp.int32),
            pltpu.SemaphoreType.DMA,
            pltpu.SemaphoreType.DMA,
        ],
    )
    def sel(x_hbm, outv_hbm, outi_hbm, buf, cv, ci, sv, si, sem0, sem1):
        wid = lax.axis_index("s") * SC_CORES + lax.axis_index("c")
        lane = lax.iota(jnp.int32, 16)

        def count_gt_win(u, nc):
            def cb(j, acc):
                for t in range(4):
                    v = cv[pl.ds((j * 4 + t) * 16, 16)]
                    valid = ((j * 4 + t) * 16 + lane) < nc
                    acc = acc + jnp.where((v > u) & valid, 1, 0)
                return acc
            acc = lax.fori_loop(0, WIN // 64, cb, jnp.zeros((16,), jnp.int32))
            return jnp.sum(acc)

        def count_gt_full(u, nc):
            def cb(j, acc):
                for t in range(4):
                    v = cv[pl.ds((j * 4 + t) * 16, 16)]
                    valid = ((j * 4 + t) * 16 + lane) < nc
                    acc = acc + jnp.where((v > u) & valid, 1, 0)
                return acc
            acc = lax.fori_loop(0, (nc + 63) // 64, cb,
                                jnp.zeros((16,), jnp.int32))
            return jnp.sum(acc)

        def bisect(nc, counter):
            def bb(_, lohi):
                lo, hi = lohi
                mid = jnp.float32(0.5) * (lo + hi)
                good = counter(mid, nc) >= K
                return (jnp.where(good, mid, lo), jnp.where(good, hi, mid))
            lo, _ = lax.fori_loop(0, BISECT_ITERS, bb,
                                  (jnp.float32(-64.0), jnp.float32(64.0)))
            return lo

        def reselect(thr, nc):
            nv = (nc + 15) // 16
            def rb(j, run):
                v = cv[pl.ds(j * 16, 16)]
                ii = ci[pl.ds(j * 16, 16)]
                valid = (j * 16 + lane) < nc
                msk = (v > thr) & valid
                mi = msk.astype(jnp.int32)
                pc = plsc.cumsum(mi)
                pos = run + (pc - mi)
                plsc.store_scatter(cv, [pos], v, mask=msk)
                plsc.store_scatter(ci, [pos], ii, mask=msk)
                return run + jnp.max(pc)
            run = lax.fori_loop(0, nv, rb, jnp.zeros((16,), jnp.int32))
            return jnp.max(run)

        def maybe_retighten(thr, nc, trig, counter):
            def yes(args):
                thr0, nc0 = args
                lo = jnp.maximum(thr0, bisect(nc0, counter))
                return lo, reselect(lo, nc0)
            return lax.cond(nc > trig, yes, lambda a: a, (thr, nc))

        def process_chunk(base_off, col0, thr, nc):
            def fb(i, run):
                for t in range(UNROLL):
                    k = i * UNROLL + t
                    v = buf[pl.ds(base_off + k * 16, 16)]
                    msk = v > thr
                    mi = msk.astype(jnp.int32)
                    pc = plsc.cumsum(mi)
                    pos = jnp.minimum(run + (pc - mi), CAP - 1)
                    plsc.store_scatter(cv, [pos], v, mask=msk)
                    plsc.store_scatter(ci, [pos], col0 + k * 16 + lane,
                                       mask=msk)
                    run = run + jnp.max(pc)
                return run
            run0 = jnp.zeros((16,), jnp.int32) + nc
            run = lax.fori_loop(0, CH // (16 * UNROLL), fb, run0)
            return jnp.max(run)

        def row_fn(q, _):
            row = wid * rows_per_w + q
            rbase = row * C
            pltpu.make_async_copy(
                x_hbm.at[pl.ds(rbase, CH)], buf.at[pl.ds(0, CH)], sem0).start()
            thr = neg_inf
            nc = jnp.int32(0)

            def pair_body(p, carry):
                thr, nc = carry
                k0 = 2 * p
                pltpu.make_async_copy(
                    x_hbm.at[pl.ds(rbase + (k0 + 1) * CH, CH)],
                    buf.at[pl.ds(CH, CH)], sem1).start()
                pltpu.make_async_copy(
                    x_hbm.at[pl.ds(rbase + k0 * CH, CH)],
                    buf.at[pl.ds(0, CH)], sem0).wait()
                nc = process_chunk(0, k0 * CH, thr, nc)
                thr, nc = maybe_retighten(thr, nc, RESEL_TRIG, count_gt_win)

                @pl.when(p < npairs - 1)
                def _():
                    pltpu.make_async_copy(
                        x_hbm.at[pl.ds(rbase + (k0 + 2) * CH, CH)],
                        buf.at[pl.ds(0, CH)], sem0).start()
                pltpu.make_async_copy(
                    x_hbm.at[pl.ds(rbase + (k0 + 1) * CH, CH)],
                    buf.at[pl.ds(CH, CH)], sem1).wait()
                nc = process_chunk(CH, (k0 + 1) * CH, thr, nc)
                thr, nc = maybe_retighten(thr, nc, RESEL_TRIG, count_gt_win)
                return thr, nc

            thr, nc = lax.fori_loop(0, npairs, pair_body, (thr, nc))
            thr, nc = maybe_retighten(thr, nc, OUTCAP, count_gt_full)
            ncf = jnp.minimum(nc, OUTCAP)
            for t in range(OUTCAP // 16):
                g = (t * 16 + lane) < ncf
                sv[pl.ds(t * 16, 16)] = jnp.where(g, cv[pl.ds(t * 16, 16)],
                                                  neg_inf)
                si[pl.ds(t * 16, 16)] = jnp.where(g, ci[pl.ds(t * 16, 16)], 0)
            pltpu.sync_copy(sv, outv_hbm.at[pl.ds(row * OUTCAP, OUTCAP)])
            pltpu.sync_copy(si, outi_hbm.at[pl.ds(row * OUTCAP, OUTCAP)])
            return 0

        lax.fori_loop(0, rows_per_w, row_fn, 0)

    fv, fi = sel(logits.reshape(-1))
    return fv.reshape(B, OUTCAP), fi.reshape(B, OUTCAP)


def _thresh_body(v_ref, i_ref, m_ref, z_ref, pc_ref, t_ref, q_ref,
                 tout_ref, bout_ref):
    v = v_ref[...]
    ii = i_ref[...]
    gt = (v[:, None, :] > v[:, :, None]) | (
        (v[:, None, :] == v[:, :, None]) & (ii[:, None, :] < ii[:, :, None]))
    rank = jnp.sum(gt.astype(jnp.int32), axis=2)
    oh = rank[:, :, None] == lax.broadcasted_iota(
        jnp.int32, (v.shape[0], OUTCAP, K), 2)
    sv = jnp.sum(jnp.where(oh, v[:, :, None], 0.0), axis=1)
    si = jnp.sum(jnp.where(oh, ii[:, :, None], 0), axis=1)

    y = sv / t_ref[...]
    s = jnp.exp(y - m_ref[...]) / z_ref[...]
    c = s
    d = 1
    while d < K:
        c = c + jnp.concatenate(
            [jnp.zeros((c.shape[0], d), c.dtype), c[:, :K - d]], axis=1)
        d *= 2
    f = c + pc_ref[...]
    cnt = jnp.sum((f <= q_ref[...]).astype(jnp.int32), axis=1, keepdims=True)
    sizes = jnp.minimum(cnt + 1, K)
    col = lax.broadcasted_iota(jnp.int32, sv.shape, 1)
    oh2 = col == (sizes - 1)
    tout_ref[...] = jnp.sum(jnp.where(oh2, sv, 0.0), axis=1, keepdims=True)
    bout_ref[...] = jnp.sum(jnp.where(oh2, si, 0), axis=1, keepdims=True)


def _mask_body(x_ref, t_ref, b_ref, mask_ref, copy_ref):
    x = x_ref[...]
    t = t_ref[...]
    b = b_ref[...]
    col = lax.broadcasted_iota(jnp.int32, x.shape, 1)
    keep = (x > t) | ((x == t) & (col <= b))
    mask_ref[...] = keep.astype(jnp.float32)
    copy_ref[...] = x


def kernel(logits, T, penalties, Qhat):
    B, C = logits.shape
    f32 = jnp.float32
    T2 = jnp.asarray(T, f32).reshape(1, 1)
    Q2 = jnp.asarray(Qhat, f32).reshape(1, 1)
    nblk = B // ROWS_PER_BLK

    m, Z = pl.pallas_call(
        _stats_body,
        grid=(nblk,),
        in_specs=[pl.BlockSpec((ROWS_PER_BLK, C), lambda i: (i, 0)),
                  pl.BlockSpec((1, 1), lambda i: (0, 0))],
        out_specs=[pl.BlockSpec((ROWS_PER_BLK, 1), lambda i: (i, 0)),
                   pl.BlockSpec((ROWS_PER_BLK, 1), lambda i: (i, 0))],
        out_shape=[jax.ShapeDtypeStruct((B, 1), f32),
                   jax.ShapeDtypeStruct((B, 1), f32)],
    )(logits, T2)

    candv, candi = _sc_select(logits)

    pcK = jnp.cumsum(penalties, axis=1)[:, :K]

    tcol, bcol = pl.pallas_call(
        _thresh_body,
        grid=(nblk,),
        in_specs=[pl.BlockSpec((ROWS_PER_BLK, OUTCAP), lambda i: (i, 0)),
                  pl.BlockSpec((ROWS_PER_BLK, OUTCAP), lambda i: (i, 0)),
                  pl.BlockSpec((ROWS_PER_BLK, 1), lambda i: (i, 0)),
                  pl.BlockSpec((ROWS_PER_BLK, 1), lambda i: (i, 0)),
                  pl.BlockSpec((1, K), lambda i: (0, 0)),
                  pl.BlockSpec((1, 1), lambda i: (0, 0)),
                  pl.BlockSpec((1, 1), lambda i: (0, 0))],
        out_specs=[pl.BlockSpec((ROWS_PER_BLK, 1), lambda i: (i, 0)),
                   pl.BlockSpec((ROWS_PER_BLK, 1), lambda i: (i, 0))],
        out_shape=[jax.ShapeDtypeStruct((B, 1), f32),
                   jax.ShapeDtypeStruct((B, 1), jnp.int32)],
    )(candv, candi, m, Z, pcK, T2, Q2)

    mask, xcopy = pl.pallas_call(
        _mask_body,
        grid=(nblk,),
        in_specs=[pl.BlockSpec((ROWS_PER_BLK, C), lambda i: (i, 0)),
                  pl.BlockSpec((ROWS_PER_BLK, 1), lambda i: (i, 0)),
                  pl.BlockSpec((ROWS_PER_BLK, 1), lambda i: (i, 0))],
        out_specs=[pl.BlockSpec((ROWS_PER_BLK, C), lambda i: (i, 0)),
                   pl.BlockSpec((ROWS_PER_BLK, C), lambda i: (i, 0))],
        out_shape=[jax.ShapeDtypeStruct((B, C), f32),
                   jax.ShapeDtypeStruct((B, C), f32)],
    )(logits, tcol, bcol)

    return (xcopy, mask)


# speedup vs baseline: 23.9775x; 1.0904x over previous
import functools

import jax
import jax.numpy as jnp
from jax import lax
from jax.experimental import pallas as pl
from jax.experimental.pallas import tpu as pltpu
from jax.experimental.pallas import tpu_sc as plsc

K = 128
ROWS_PER_BLK = 8

SC_CORES = 2
SC_SUBCORES = 16
NW = SC_CORES * SC_SUBCORES
CH = 10000
CAP = 26624
OUTCAP = 256
RESEL_TRIG = 4096
BISECT_ITERS = 26
WIN = 2048
UNROLL = 5


def _stats_body(x_ref, t_ref, m_ref, z_ref):
    x = x_ref[...]
    y = x / t_ref[...]
    m = jnp.max(y, axis=1, keepdims=True)
    e = jnp.exp(y - m)
    z = jnp.sum(e, axis=1, keepdims=True)
    m_ref[...] = m
    z_ref[...] = z


def _sc_select(logits):
    B, C = logits.shape
    rows_per_w = B // NW
    npairs = C // (2 * CH)
    mesh = plsc.VectorSubcoreMesh(
        core_axis_name="c", subcore_axis_name="s",
        num_cores=SC_CORES, num_subcores=SC_SUBCORES)
    neg_inf = jnp.float32(-jnp.inf)

    @functools.partial(
        pl.kernel,
        out_type=[jax.ShapeDtypeStruct((B * OUTCAP,), jnp.float32),
                  jax.ShapeDtypeStruct((B * OUTCAP,), jnp.int32)],
        mesh=mesh,
        compiler_params=pltpu.CompilerParams(needs_layout_passes=False),
        scratch_types=[
            pltpu.VMEM((2 * CH,), jnp.float32),
            pltpu.VMEM((CAP,), jnp.float32),
            pltpu.VMEM((CAP,), jnp.int32),
            pltpu.VMEM((OUTCAP,), jnp.float32),
            pltpu.VMEM((OUTCAP,), jnp.int32),
            pltpu.SemaphoreType.DMA,
            pltpu.SemaphoreType.DMA,
        ],
    )
    def sel(x_hbm, outv_hbm, outi_hbm, buf, cv, ci, sv, si, sem0, sem1):
        wid = lax.axis_index("s") * SC_CORES + lax.axis_index("c")
        lane = lax.iota(jnp.int32, 16)

        def count_gt_win(u, nc):
            def cb(j, acc):
                for t in range(4):
                    v = cv[pl.ds((j * 4 + t) * 16, 16)]
                    valid = ((j * 4 + t) * 16 + lane) < nc
                    acc = acc + jnp.where((v > u) & valid, 1, 0)
                return acc
            acc = lax.fori_loop(0, WIN // 64, cb, jnp.zeros((16,), jnp.int32))
            return jnp.sum(acc)

        def count_gt_full(u, nc):
            def cb(j, acc):
                for t in range(4):
                    v = cv[pl.ds((j * 4 + t) * 16, 16)]
                    valid = ((j * 4 + t) * 16 + lane) < nc
                    acc = acc + jnp.where((v > u) & valid, 1, 0)
                return acc
            acc = lax.fori_loop(0, (nc + 63) // 64, cb,
                                jnp.zeros((16,), jnp.int32))
            return jnp.sum(acc)

        def bisect(nc, counter):
            def bb(_, lohi):
                lo, hi = lohi
                mid = jnp.float32(0.5) * (lo + hi)
                good = counter(mid, nc) >= K
                return (jnp.where(good, mid, lo), jnp.where(good, hi, mid))
            lo, _ = lax.fori_loop(0, BISECT_ITERS, bb,
                                  (jnp.float32(-64.0), jnp.float32(64.0)))
            return lo

        def reselect(thr, nc):
            nv = (nc + 15) // 16
            def rb(j, run):
                v = cv[pl.ds(j * 16, 16)]
                ii = ci[pl.ds(j * 16, 16)]
                valid = (j * 16 + lane) < nc
                msk = (v > thr) & valid
                mi = msk.astype(jnp.int32)
                pc = plsc.cumsum(mi)
                pos = run + (pc - mi)
                plsc.store_scatter(cv, [pos], v, mask=msk)
                plsc.store_scatter(ci, [pos], ii, mask=msk)
                return run + plsc.all_reduce_population_count(msk)
            run = lax.fori_loop(0, nv, rb, jnp.zeros((16,), jnp.int32))
            return jnp.max(run)

        def buf_bisect(thr):
            def cg(u):
                def cb(j, acc):
                    for t in range(4):
                        v = buf[pl.ds((j * 4 + t) * 16, 16)]
                        acc = acc + jnp.where(v > u, 1, 0)
                    return acc
                acc = lax.fori_loop(0, WIN // 64, cb,
                                    jnp.zeros((16,), jnp.int32))
                return jnp.sum(acc)
            def bb(_, lohi):
                lo, hi = lohi
                mid = jnp.float32(0.5) * (lo + hi)
                good = cg(mid) >= K
                return (jnp.where(good, mid, lo), jnp.where(good, hi, mid))
            lo, _ = lax.fori_loop(0, BISECT_ITERS, bb,
                                  (jnp.float32(-64.0), jnp.float32(64.0)))
            return jnp.maximum(thr, lo)

        def maybe_retighten(thr, nc, trig, counter):
            def yes(args):
                thr0, nc0 = args
                lo = jnp.maximum(thr0, bisect(nc0, counter))
                return lo, reselect(lo, nc0)
            return lax.cond(nc > trig, yes, lambda a: a, (thr, nc))

        def process_chunk(base_off, col0, thr, nc):
            def fb(i, run):
                for t in range(UNROLL):
                    k = i * UNROLL + t
                    v = buf[pl.ds(base_off + k * 16, 16)]
                    msk = v > thr
                    mi = msk.astype(jnp.int32)
                    pc = plsc.cumsum(mi)
                    pos = jnp.minimum(run + (pc - mi), CAP - 1)
                    plsc.store_scatter(cv, [pos], v, mask=msk)
                    plsc.store_scatter(ci, [pos], col0 + k * 16 + lane,
                                       mask=msk)
                    run = run + plsc.all_reduce_population_count(msk)
                return run
            run0 = jnp.zeros((16,), jnp.int32) + nc
            run = lax.fori_loop(0, CH // (16 * UNROLL), fb, run0)
            return jnp.max(run)

        def row_fn(q, _):
            row = wid * rows_per_w + q
            rbase = row * C
            pltpu.make_async_copy(
                x_hbm.at[pl.ds(rbase, CH)], buf.at[pl.ds(0, CH)], sem0).start()
            thr = neg_inf
            nc = jnp.int32(0)

            def pair_body(p, carry):
                thr, nc = carry
                k0 = 2 * p
                pltpu.make_async_copy(
                    x_hbm.at[pl.ds(rbase + (k0 + 1) * CH, CH)],
                    buf.at[pl.ds(CH, CH)], sem1).start()
                pltpu.make_async_copy(
                    x_hbm.at[pl.ds(rbase + k0 * CH, CH)],
                    buf.at[pl.ds(0, CH)], sem0).wait()
                thr = lax.cond(p == 0, buf_bisect, lambda t: t, thr)
                nc = process_chunk(0, k0 * CH, thr, nc)
                thr, nc = maybe_retighten(thr, nc, RESEL_TRIG, count_gt_win)

                @pl.when(p < npairs - 1)
                def _():
                    pltpu.make_async_copy(
                        x_hbm.at[pl.ds(rbase + (k0 + 2) * CH, CH)],
                        buf.at[pl.ds(0, CH)], sem0).start()
                pltpu.make_async_copy(
                    x_hbm.at[pl.ds(rbase + (k0 + 1) * CH, CH)],
                    buf.at[pl.ds(CH, CH)], sem1).wait()
                nc = process_chunk(CH, (k0 + 1) * CH, thr, nc)
                thr, nc = maybe_retighten(thr, nc, RESEL_TRIG, count_gt_win)
                return thr, nc

            thr, nc = lax.fori_loop(0, npairs, pair_body, (thr, nc))
            thr, nc = maybe_retighten(thr, nc, OUTCAP, count_gt_full)
            ncf = jnp.minimum(nc, OUTCAP)
            for t in range(OUTCAP // 16):
                g = (t * 16 + lane) < ncf
                sv[pl.ds(t * 16, 16)] = jnp.where(g, cv[pl.ds(t * 16, 16)],
                                                  neg_inf)
                si[pl.ds(t * 16, 16)] = jnp.where(g, ci[pl.ds(t * 16, 16)], 0)
            pltpu.sync_copy(sv, outv_hbm.at[pl.ds(row * OUTCAP, OUTCAP)])
            pltpu.sync_copy(si, outi_hbm.at[pl.ds(row * OUTCAP, OUTCAP)])
            return 0

        lax.fori_loop(0, rows_per_w, row_fn, 0)

    fv, fi = sel(logits.reshape(-1))
    return fv.reshape(B, OUTCAP), fi.reshape(B, OUTCAP)


def _thresh_body(v_ref, i_ref, m_ref, z_ref, pc_ref, t_ref, q_ref,
                 tout_ref, bout_ref):
    v = v_ref[...]
    ii = i_ref[...]
    gt = (v[:, None, :] > v[:, :, None]) | (
        (v[:, None, :] == v[:, :, None]) & (ii[:, None, :] < ii[:, :, None]))
    rank = jnp.sum(gt.astype(jnp.int32), axis=2)
    oh = rank[:, :, None] == lax.broadcasted_iota(
        jnp.int32, (v.shape[0], OUTCAP, K), 2)
    sv = jnp.sum(jnp.where(oh, v[:, :, None], 0.0), axis=1)
    si = jnp.sum(jnp.where(oh, ii[:, :, None], 0), axis=1)

    y = sv / t_ref[...]
    s = jnp.exp(y - m_ref[...]) / z_ref[...]
    c = s
    d = 1
    while d < K:
        c = c + jnp.concatenate(
            [jnp.zeros((c.shape[0], d), c.dtype), c[:, :K - d]], axis=1)
        d *= 2
    f = c + pc_ref[...]
    cnt = jnp.sum((f <= q_ref[...]).astype(jnp.int32), axis=1, keepdims=True)
    sizes = jnp.minimum(cnt + 1, K)
    col = lax.broadcasted_iota(jnp.int32, sv.shape, 1)
    oh2 = col == (sizes - 1)
    tout_ref[...] = jnp.sum(jnp.where(oh2, sv, 0.0), axis=1, keepdims=True)
    bout_ref[...] = jnp.sum(jnp.where(oh2, si, 0), axis=1, keepdims=True)


def _mask_body(x_ref, t_ref, b_ref, mask_ref, copy_ref):
    x = x_ref[...]
    t = t_ref[...]
    b = b_ref[...]
    col = lax.broadcasted_iota(jnp.int32, x.shape, 1)
    keep = (x > t) | ((x == t) & (col <= b))
    mask_ref[...] = keep.astype(jnp.float32)
    copy_ref[...] = x


def kernel(logits, T, penalties, Qhat):
    B, C = logits.shape
    f32 = jnp.float32
    T2 = jnp.asarray(T, f32).reshape(1, 1)
    Q2 = jnp.asarray(Qhat, f32).reshape(1, 1)
    nblk = B // ROWS_PER_BLK

    m, Z = pl.pallas_call(
        _stats_body,
        grid=(nblk,),
        in_specs=[pl.BlockSpec((ROWS_PER_BLK, C), lambda i: (i, 0)),
                  pl.BlockSpec((1, 1), lambda i: (0, 0))],
        out_specs=[pl.BlockSpec((ROWS_PER_BLK, 1), lambda i: (i, 0)),
                   pl.BlockSpec((ROWS_PER_BLK, 1), lambda i: (i, 0))],
        out_shape=[jax.ShapeDtypeStruct((B, 1), f32),
                   jax.ShapeDtypeStruct((B, 1), f32)],
    )(logits, T2)

    candv, candi = _sc_select(logits)

    pcK = jnp.cumsum(penalties, axis=1)[:, :K]

    tcol, bcol = pl.pallas_call(
        _thresh_body,
        grid=(nblk,),
        in_specs=[pl.BlockSpec((ROWS_PER_BLK, OUTCAP), lambda i: (i, 0)),
                  pl.BlockSpec((ROWS_PER_BLK, OUTCAP), lambda i: (i, 0)),
                  pl.BlockSpec((ROWS_PER_BLK, 1), lambda i: (i, 0)),
                  pl.BlockSpec((ROWS_PER_BLK, 1), lambda i: (i, 0)),
                  pl.BlockSpec((1, K), lambda i: (0, 0)),
                  pl.BlockSpec((1, 1), lambda i: (0, 0)),
                  pl.BlockSpec((1, 1), lambda i: (0, 0))],
        out_specs=[pl.BlockSpec((ROWS_PER_BLK, 1), lambda i: (i, 0)),
                   pl.BlockSpec((ROWS_PER_BLK, 1), lambda i: (i, 0))],
        out_shape=[jax.ShapeDtypeStruct((B, 1), f32),
                   jax.ShapeDtypeStruct((B, 1), jnp.int32)],
    )(candv, candi, m, Z, pcK, T2, Q2)

    mask, xcopy = pl.pallas_call(
        _mask_body,
        grid=(nblk,),
        in_specs=[pl.BlockSpec((ROWS_PER_BLK, C), lambda i: (i, 0)),
                  pl.BlockSpec((ROWS_PER_BLK, 1), lambda i: (i, 0)),
                  pl.BlockSpec((ROWS_PER_BLK, 1), lambda i: (i, 0))],
        out_specs=[pl.BlockSpec((ROWS_PER_BLK, C), lambda i: (i, 0)),
                   pl.BlockSpec((ROWS_PER_BLK, C), lambda i: (i, 0))],
        out_shape=[jax.ShapeDtypeStruct((B, C), f32),
                   jax.ShapeDtypeStruct((B, C), f32)],
    )(logits, tcol, bcol)

    return (xcopy, mask)


# speedup vs baseline: 24.0207x; 1.0018x over previous
import functools

import jax
import jax.numpy as jnp
from jax import lax
from jax.experimental import pallas as pl
from jax.experimental.pallas import tpu as pltpu
from jax.experimental.pallas import tpu_sc as plsc

K = 128
ROWS_PER_BLK = 8

SC_CORES = 2
SC_SUBCORES = 16
NW = SC_CORES * SC_SUBCORES
CH = 10000
CAP = 26624
OUTCAP = 256
RESEL_TRIG = 4096
BISECT_ITERS = 26
WIN = 2048
UNROLL = 5


def _stats_body(x_ref, t_ref, m_ref, z_ref):
    x = x_ref[...]
    y = x / t_ref[...]
    m = jnp.max(y, axis=1, keepdims=True)
    e = jnp.exp(y - m)
    z = jnp.sum(e, axis=1, keepdims=True)
    m_ref[...] = m
    z_ref[...] = z


def _sc_select(logits):
    B, C = logits.shape
    rows_per_w = B // NW
    npairs = C // (2 * CH)
    mesh = plsc.VectorSubcoreMesh(
        core_axis_name="c", subcore_axis_name="s",
        num_cores=SC_CORES, num_subcores=SC_SUBCORES)
    neg_inf = jnp.float32(-jnp.inf)

    @functools.partial(
        pl.kernel,
        out_type=[jax.ShapeDtypeStruct((B * OUTCAP,), jnp.float32),
                  jax.ShapeDtypeStruct((B * OUTCAP,), jnp.int32)],
        mesh=mesh,
        compiler_params=pltpu.CompilerParams(needs_layout_passes=False),
        scratch_types=[
            pltpu.VMEM((2 * CH,), jnp.float32),
            pltpu.VMEM((CAP,), jnp.float32),
            pltpu.VMEM((CAP,), jnp.int32),
            pltpu.VMEM((OUTCAP,), jnp.float32),
            pltpu.VMEM((OUTCAP,), jnp.int32),
            pltpu.SemaphoreType.DMA,
            pltpu.SemaphoreType.DMA,
        ],
    )
    def sel(x_hbm, outv_hbm, outi_hbm, buf, cv, ci, sv, si, sem0, sem1):
        wid = lax.axis_index("s") * SC_CORES + lax.axis_index("c")
        lane = lax.iota(jnp.int32, 16)

        def count_gt_win(u, nc):
            def cb(j, acc):
                for t in range(4):
                    v = cv[pl.ds((j * 4 + t) * 16, 16)]
                    valid = ((j * 4 + t) * 16 + lane) < nc
                    acc = acc + jnp.where((v > u) & valid, 1, 0)
                return acc
            acc = lax.fori_loop(0, WIN // 64, cb, jnp.zeros((16,), jnp.int32))
            return jnp.sum(acc)

        def count_gt_full(u, nc):
            def cb(j, acc):
                for t in range(4):
                    v = cv[pl.ds((j * 4 + t) * 16, 16)]
                    valid = ((j * 4 + t) * 16 + lane) < nc
                    acc = acc + jnp.where((v > u) & valid, 1, 0)
                return acc
            acc = lax.fori_loop(0, (nc + 63) // 64, cb,
                                jnp.zeros((16,), jnp.int32))
            return jnp.sum(acc)

        def bisect(nc, counter):
            def bb(_, lohi):
                lo, hi = lohi
                mid = jnp.float32(0.5) * (lo + hi)
                good = counter(mid, nc) >= K
                return (jnp.where(good, mid, lo), jnp.where(good, hi, mid))
            lo, _ = lax.fori_loop(0, BISECT_ITERS, bb,
                                  (jnp.float32(-64.0), jnp.float32(64.0)))
            return lo

        def reselect(thr, nc):
            nv = (nc + 15) // 16
            def rb(j, run):
                v = cv[pl.ds(j * 16, 16)]
                ii = ci[pl.ds(j * 16, 16)]
                valid = (j * 16 + lane) < nc
                msk = (v > thr) & valid
                mi = msk.astype(jnp.int32)
                pc = plsc.cumsum(mi)
                pos = run + (pc - mi)
                plsc.store_scatter(cv, [pos], v, mask=msk)
                plsc.store_scatter(ci, [pos], ii, mask=msk)
                return run + plsc.all_reduce_population_count(msk)
            run = lax.fori_loop(0, nv, rb, jnp.zeros((16,), jnp.int32))
            return jnp.max(run)

        def buf_bisect(thr):
            def cg(u):
                def cb(j, acc):
                    for t in range(4):
                        v = buf[pl.ds((j * 4 + t) * 16, 16)]
                        acc = acc + jnp.where(v > u, 1, 0)
                    return acc
                acc = lax.fori_loop(0, WIN // 64, cb,
                                    jnp.zeros((16,), jnp.int32))
                return jnp.sum(acc)
            def bb(_, lohi):
                lo, hi = lohi
                mid = jnp.float32(0.5) * (lo + hi)
                good = cg(mid) >= K
                return (jnp.where(good, mid, lo), jnp.where(good, hi, mid))
            lo, _ = lax.fori_loop(0, BISECT_ITERS, bb,
                                  (jnp.float32(-64.0), jnp.float32(64.0)))
            return jnp.maximum(thr, lo)

        def maybe_retighten(thr, nc, trig, counter):
            def yes(args):
                thr0, nc0 = args
                lo = jnp.maximum(thr0, bisect(nc0, counter))
                return lo, reselect(lo, nc0)
            return lax.cond(nc > trig, yes, lambda a: a, (thr, nc))

        def process_chunk(base_off, col0, thr, nc):
            def fb(i, run):
                for t in range(UNROLL):
                    k = i * UNROLL + t
                    v = buf[pl.ds(base_off + k * 16, 16)]
                    msk = v > thr
                    mi = msk.astype(jnp.int32)
                    pc = plsc.cumsum(mi)
                    pos = jnp.minimum(run + (pc - mi), CAP - 1)
                    plsc.store_scatter(cv, [pos], v, mask=msk)
                    plsc.store_scatter(ci, [pos], col0 + k * 16 + lane,
                                       mask=msk)
                    run = run + plsc.all_reduce_population_count(msk)
                return run
            run0 = jnp.zeros((16,), jnp.int32) + nc
            run = lax.fori_loop(0, CH // (16 * UNROLL), fb, run0)
            return jnp.max(run)

        def row_fn(q, _):
            row = wid * rows_per_w + q
            rbase = row * C
            pltpu.make_async_copy(
                x_hbm.at[pl.ds(rbase, CH)], buf.at[pl.ds(0, CH)], sem0).start()
            thr = neg_inf
            nc = jnp.int32(0)

            def pair_body(p, carry):
                thr, nc = carry
                k0 = 2 * p
                pltpu.make_async_copy(
                    x_hbm.at[pl.ds(rbase + (k0 + 1) * CH, CH)],
                    buf.at[pl.ds(CH, CH)], sem1).start()
                pltpu.make_async_copy(
                    x_hbm.at[pl.ds(rbase + k0 * CH, CH)],
                    buf.at[pl.ds(0, CH)], sem0).wait()
                thr = lax.cond(p == 0, buf_bisect, lambda t: t, thr)
                nc = process_chunk(0, k0 * CH, thr, nc)
                thr, nc = maybe_retighten(thr, nc, RESEL_TRIG, count_gt_win)

                @pl.when(p < npairs - 1)
                def _():
                    pltpu.make_async_copy(
                        x_hbm.at[pl.ds(rbase + (k0 + 2) * CH, CH)],
                        buf.at[pl.ds(0, CH)], sem0).start()
                pltpu.make_async_copy(
                    x_hbm.at[pl.ds(rbase + (k0 + 1) * CH, CH)],
                    buf.at[pl.ds(CH, CH)], sem1).wait()
                nc = process_chunk(CH, (k0 + 1) * CH, thr, nc)
                thr, nc = maybe_retighten(thr, nc, RESEL_TRIG, count_gt_win)
                return thr, nc

            thr, nc = lax.fori_loop(0, npairs, pair_body, (thr, nc))
            thr, nc = maybe_retighten(thr, nc, OUTCAP, count_gt_full)
            ncf = jnp.minimum(nc, OUTCAP)
            for t in range(OUTCAP // 16):
                g = (t * 16 + lane) < ncf
                sv[pl.ds(t * 16, 16)] = jnp.where(g, cv[pl.ds(t * 16, 16)],
                                                  neg_inf)
                si[pl.ds(t * 16, 16)] = jnp.where(g, ci[pl.ds(t * 16, 16)], 0)
            pltpu.sync_copy(sv, outv_hbm.at[pl.ds(row * OUTCAP, OUTCAP)])
            pltpu.sync_copy(si, outi_hbm.at[pl.ds(row * OUTCAP, OUTCAP)])
            return 0

        lax.fori_loop(0, rows_per_w, row_fn, 0)

    fv, fi = sel(logits.reshape(-1))
    return fv.reshape(B, OUTCAP), fi.reshape(B, OUTCAP)


def _fused_body(x_ref, v_ref, i_ref, pc_ref, t_ref, q_ref,
                mask_ref, copy_ref):
    x = x_ref[...]
    yx = x / t_ref[...]
    m_ref_v = jnp.max(yx, axis=1, keepdims=True)
    z_ref_v = jnp.sum(jnp.exp(yx - m_ref_v), axis=1, keepdims=True)

    v = v_ref[...]
    ii = i_ref[...]
    gt = (v[:, None, :] > v[:, :, None]) | (
        (v[:, None, :] == v[:, :, None]) & (ii[:, None, :] < ii[:, :, None]))
    rank = jnp.sum(gt.astype(jnp.int32), axis=2)
    oh = rank[:, :, None] == lax.broadcasted_iota(
        jnp.int32, (v.shape[0], OUTCAP, K), 2)
    sv = jnp.sum(jnp.where(oh, v[:, :, None], 0.0), axis=1)
    si = jnp.sum(jnp.where(oh, ii[:, :, None], 0), axis=1)

    y = sv / t_ref[...]
    s = jnp.exp(y - m_ref_v) / z_ref_v
    c = s
    d = 1
    while d < K:
        c = c + jnp.concatenate(
            [jnp.zeros((c.shape[0], d), c.dtype), c[:, :K - d]], axis=1)
        d *= 2
    f = c + pc_ref[...]
    cnt = jnp.sum((f <= q_ref[...]).astype(jnp.int32), axis=1, keepdims=True)
    sizes = jnp.minimum(cnt + 1, K)
    col = lax.broadcasted_iota(jnp.int32, sv.shape, 1)
    oh2 = col == (sizes - 1)
    t = jnp.sum(jnp.where(oh2, sv, 0.0), axis=1, keepdims=True)
    b = jnp.sum(jnp.where(oh2, si, 0), axis=1, keepdims=True)

    colx = lax.broadcasted_iota(jnp.int32, x.shape, 1)
    keep = (x > t) | ((x == t) & (colx <= b))
    mask_ref[...] = keep.astype(jnp.float32)
    copy_ref[...] = x


def kernel(logits, T, penalties, Qhat):
    B, C = logits.shape
    f32 = jnp.float32
    T2 = jnp.asarray(T, f32).reshape(1, 1)
    Q2 = jnp.asarray(Qhat, f32).reshape(1, 1)
    nblk = B // ROWS_PER_BLK

    candv, candi = _sc_select(logits)

    pcK = jnp.cumsum(penalties, axis=1)[:, :K]

    mask, xcopy = pl.pallas_call(
        _fused_body,
        grid=(nblk,),
        in_specs=[pl.BlockSpec((ROWS_PER_BLK, C), lambda i: (i, 0)),
                  pl.BlockSpec((ROWS_PER_BLK, OUTCAP), lambda i: (i, 0)),
                  pl.BlockSpec((ROWS_PER_BLK, OUTCAP), lambda i: (i, 0)),
                  pl.BlockSpec((1, K), lambda i: (0, 0)),
                  pl.BlockSpec((1, 1), lambda i: (0, 0)),
                  pl.BlockSpec((1, 1), lambda i: (0, 0))],
        out_specs=[pl.BlockSpec((ROWS_PER_BLK, C), lambda i: (i, 0)),
                   pl.BlockSpec((ROWS_PER_BLK, C), lambda i: (i, 0))],
        out_shape=[jax.ShapeDtypeStruct((B, C), f32),
                   jax.ShapeDtypeStruct((B, C), f32)],
    )(logits, candv, candi, pcK, T2, Q2)

    return (xcopy, mask)


# speedup vs baseline: 24.3476x; 1.0136x over previous
import functools

import jax
import jax.numpy as jnp
from jax import lax
from jax.experimental import pallas as pl
from jax.experimental.pallas import tpu as pltpu
from jax.experimental.pallas import tpu_sc as plsc

K = 128
ROWS_PER_BLK = 8

SC_CORES = 2
SC_SUBCORES = 16
NW = SC_CORES * SC_SUBCORES
CH = 10000
CAP = 26624
OUTCAP = 256
RESEL_TRIG = 4096
BISECT_ITERS = 21
WIN = 2048
UNROLL = 10


def _stats_body(x_ref, t_ref, m_ref, z_ref):
    x = x_ref[...]
    y = x / t_ref[...]
    m = jnp.max(y, axis=1, keepdims=True)
    e = jnp.exp(y - m)
    z = jnp.sum(e, axis=1, keepdims=True)
    m_ref[...] = m
    z_ref[...] = z


def _sc_select(logits):
    B, C = logits.shape
    rows_per_w = B // NW
    npairs = C // (2 * CH)
    mesh = plsc.VectorSubcoreMesh(
        core_axis_name="c", subcore_axis_name="s",
        num_cores=SC_CORES, num_subcores=SC_SUBCORES)
    neg_inf = jnp.float32(-jnp.inf)

    @functools.partial(
        pl.kernel,
        out_type=[jax.ShapeDtypeStruct((B * OUTCAP,), jnp.float32),
                  jax.ShapeDtypeStruct((B * OUTCAP,), jnp.int32)],
        mesh=mesh,
        compiler_params=pltpu.CompilerParams(needs_layout_passes=False),
        scratch_types=[
            pltpu.VMEM((2 * CH,), jnp.float32),
            pltpu.VMEM((CAP,), jnp.float32),
            pltpu.VMEM((CAP,), jnp.int32),
            pltpu.VMEM((OUTCAP,), jnp.float32),
            pltpu.VMEM((OUTCAP,), jnp.int32),
            pltpu.SemaphoreType.DMA,
            pltpu.SemaphoreType.DMA,
        ],
    )
    def sel(x_hbm, outv_hbm, outi_hbm, buf, cv, ci, sv, si, sem0, sem1):
        wid = lax.axis_index("s") * SC_CORES + lax.axis_index("c")
        lane = lax.iota(jnp.int32, 16)

        def count_gt_win(u, nc):
            def cb(j, acc):
                for t in range(4):
                    v = cv[pl.ds((j * 4 + t) * 16, 16)]
                    valid = ((j * 4 + t) * 16 + lane) < nc
                    acc = acc + jnp.where((v > u) & valid, 1, 0)
                return acc
            acc = lax.fori_loop(0, WIN // 64, cb, jnp.zeros((16,), jnp.int32))
            return jnp.sum(acc)

        def count_gt_full(u, nc):
            def cb(j, acc):
                for t in range(4):
                    v = cv[pl.ds((j * 4 + t) * 16, 16)]
                    valid = ((j * 4 + t) * 16 + lane) < nc
                    acc = acc + jnp.where((v > u) & valid, 1, 0)
                return acc
            acc = lax.fori_loop(0, (nc + 63) // 64, cb,
                                jnp.zeros((16,), jnp.int32))
            return jnp.sum(acc)

        def bisect(nc, counter):
            def bb(_, lohi):
                lo, hi = lohi
                mid = jnp.float32(0.5) * (lo + hi)
                good = counter(mid, nc) >= K
                return (jnp.where(good, mid, lo), jnp.where(good, hi, mid))
            lo, _ = lax.fori_loop(0, BISECT_ITERS, bb,
                                  (jnp.float32(-64.0), jnp.float32(64.0)))
            return lo

        def reselect(thr, nc):
            nv = (nc + 15) // 16
            def rb(j, run):
                v = cv[pl.ds(j * 16, 16)]
                ii = ci[pl.ds(j * 16, 16)]
                valid = (j * 16 + lane) < nc
                msk = (v > thr) & valid
                mi = msk.astype(jnp.int32)
                pc = plsc.cumsum(mi)
                pos = run + (pc - mi)
                plsc.store_scatter(cv, [pos], v, mask=msk)
                plsc.store_scatter(ci, [pos], ii, mask=msk)
                return run + plsc.all_reduce_population_count(msk)
            run = lax.fori_loop(0, nv, rb, jnp.zeros((16,), jnp.int32))
            return jnp.max(run)

        def buf_bisect(thr):
            def cg(u):
                def cb(j, acc):
                    for t in range(4):
                        v = buf[pl.ds((j * 4 + t) * 16, 16)]
                        acc = acc + jnp.where(v > u, 1, 0)
                    return acc
                acc = lax.fori_loop(0, WIN // 64, cb,
                                    jnp.zeros((16,), jnp.int32))
                return jnp.sum(acc)
            def bb(_, lohi):
                lo, hi = lohi
                mid = jnp.float32(0.5) * (lo + hi)
                good = cg(mid) >= K
                return (jnp.where(good, mid, lo), jnp.where(good, hi, mid))
            lo, _ = lax.fori_loop(0, BISECT_ITERS, bb,
                                  (jnp.float32(-64.0), jnp.float32(64.0)))
            return jnp.maximum(thr, lo)

        def maybe_retighten(thr, nc, trig, counter):
            def yes(args):
                thr0, nc0 = args
                lo = jnp.maximum(thr0, bisect(nc0, counter))
                return lo, reselect(lo, nc0)
            return lax.cond(nc > trig, yes, lambda a: a, (thr, nc))

        def process_chunk(base_off, col0, thr, nc):
            def fb(i, run):
                for t in range(UNROLL):
                    k = i * UNROLL + t
                    v = buf[pl.ds(base_off + k * 16, 16)]
                    msk = v > thr
                    mi = msk.astype(jnp.int32)
                    pc = plsc.cumsum(mi)
                    pos = jnp.minimum(run + (pc - mi), CAP - 1)
                    plsc.store_scatter(cv, [pos], v, mask=msk)
                    plsc.store_scatter(ci, [pos], col0 + k * 16 + lane,
                                       mask=msk)
                    run = run + plsc.all_reduce_population_count(msk)
                return run
            run0 = jnp.zeros((16,), jnp.int32) + nc
            run = lax.fori_loop(0, CH // (16 * UNROLL), fb, run0)
            return jnp.max(run)

        def row_fn(q, _):
            row = wid * rows_per_w + q
            rbase = row * C
            pltpu.make_async_copy(
                x_hbm.at[pl.ds(rbase, CH)], buf.at[pl.ds(0, CH)], sem0).start()
            thr = neg_inf
            nc = jnp.int32(0)

            def pair_body(p, carry):
                thr, nc = carry
                k0 = 2 * p
                pltpu.make_async_copy(
                    x_hbm.at[pl.ds(rbase + (k0 + 1) * CH, CH)],
                    buf.at[pl.ds(CH, CH)], sem1).start()
                pltpu.make_async_copy(
                    x_hbm.at[pl.ds(rbase + k0 * CH, CH)],
                    buf.at[pl.ds(0, CH)], sem0).wait()
                thr = lax.cond(p == 0, buf_bisect, lambda t: t, thr)
                nc = process_chunk(0, k0 * CH, thr, nc)
                thr, nc = maybe_retighten(thr, nc, RESEL_TRIG, count_gt_win)

                @pl.when(p < npairs - 1)
                def _():
                    pltpu.make_async_copy(
                        x_hbm.at[pl.ds(rbase + (k0 + 2) * CH, CH)],
                        buf.at[pl.ds(0, CH)], sem0).start()
                pltpu.make_async_copy(
                    x_hbm.at[pl.ds(rbase + (k0 + 1) * CH, CH)],
                    buf.at[pl.ds(CH, CH)], sem1).wait()
                nc = process_chunk(CH, (k0 + 1) * CH, thr, nc)
                thr, nc = maybe_retighten(thr, nc, RESEL_TRIG, count_gt_win)
                return thr, nc

            thr, nc = lax.fori_loop(0, npairs, pair_body, (thr, nc))
            thr, nc = maybe_retighten(thr, nc, OUTCAP, count_gt_full)
            ncf = jnp.minimum(nc, OUTCAP)
            for t in range(OUTCAP // 16):
                g = (t * 16 + lane) < ncf
                sv[pl.ds(t * 16, 16)] = jnp.where(g, cv[pl.ds(t * 16, 16)],
                                                  neg_inf)
                si[pl.ds(t * 16, 16)] = jnp.where(g, ci[pl.ds(t * 16, 16)], 0)
            pltpu.sync_copy(sv, outv_hbm.at[pl.ds(row * OUTCAP, OUTCAP)])
            pltpu.sync_copy(si, outi_hbm.at[pl.ds(row * OUTCAP, OUTCAP)])
            return 0

        lax.fori_loop(0, rows_per_w, row_fn, 0)

    fv, fi = sel(logits.reshape(-1))
    return fv.reshape(B, OUTCAP), fi.reshape(B, OUTCAP)


def _fused_body(x_ref, v_ref, i_ref, pc_ref, t_ref, q_ref,
                mask_ref, copy_ref):
    x = x_ref[...]
    yx = x / t_ref[...]
    m_ref_v = jnp.max(yx, axis=1, keepdims=True)
    z_ref_v = jnp.sum(jnp.exp(yx - m_ref_v), axis=1, keepdims=True)

    v = v_ref[...]
    ii = i_ref[...]
    gt = (v[:, None, :] > v[:, :, None]) | (
        (v[:, None, :] == v[:, :, None]) & (ii[:, None, :] < ii[:, :, None]))
    rank = jnp.sum(gt.astype(jnp.int32), axis=2)
    oh = rank[:, :, None] == lax.broadcasted_iota(
        jnp.int32, (v.shape[0], OUTCAP, K), 2)
    sv = jnp.sum(jnp.where(oh, v[:, :, None], 0.0), axis=1)
    si = jnp.sum(jnp.where(oh, ii[:, :, None], 0), axis=1)

    y = sv / t_ref[...]
    s = jnp.exp(y - m_ref_v) / z_ref_v
    c = s
    d = 1
    while d < K:
        c = c + jnp.concatenate(
            [jnp.zeros((c.shape[0], d), c.dtype), c[:, :K - d]], axis=1)
        d *= 2
    f = c + pc_ref[...]
    cnt = jnp.sum((f <= q_ref[...]).astype(jnp.int32), axis=1, keepdims=True)
    sizes = jnp.minimum(cnt + 1, K)
    col = lax.broadcasted_iota(jnp.int32, sv.shape, 1)
    oh2 = col == (sizes - 1)
    t = jnp.sum(jnp.where(oh2, sv, 0.0), axis=1, keepdims=True)
    b = jnp.sum(jnp.where(oh2, si, 0), axis=1, keepdims=True)

    colx = lax.broadcasted_iota(jnp.int32, x.shape, 1)
    keep = (x > t) | ((x == t) & (colx <= b))
    mask_ref[...] = keep.astype(jnp.float32)
    copy_ref[...] = x


def kernel(logits, T, penalties, Qhat):
    B, C = logits.shape
    f32 = jnp.float32
    T2 = jnp.asarray(T, f32).reshape(1, 1)
    Q2 = jnp.asarray(Qhat, f32).reshape(1, 1)
    nblk = B // ROWS_PER_BLK

    candv, candi = _sc_select(logits)

    pcK = jnp.cumsum(penalties, axis=1)[:, :K]

    mask, xcopy = pl.pallas_call(
        _fused_body,
        grid=(nblk,),
        in_specs=[pl.BlockSpec((ROWS_PER_BLK, C), lambda i: (i, 0)),
                  pl.BlockSpec((ROWS_PER_BLK, OUTCAP), lambda i: (i, 0)),
                  pl.BlockSpec((ROWS_PER_BLK, OUTCAP), lambda i: (i, 0)),
                  pl.BlockSpec((1, K), lambda i: (0, 0)),
                  pl.BlockSpec((1, 1), lambda i: (0, 0)),
                  pl.BlockSpec((1, 1), lambda i: (0, 0))],
        out_specs=[pl.BlockSpec((ROWS_PER_BLK, C), lambda i: (i, 0)),
                   pl.BlockSpec((ROWS_PER_BLK, C), lambda i: (i, 0))],
        out_shape=[jax.ShapeDtypeStruct((B, C), f32),
                   jax.ShapeDtypeStruct((B, C), f32)],
    )(logits, candv, candi, pcK, T2, Q2)

    return (xcopy, mask)
